# baseline (speedup 1.0000x reference)
import jax
import jax.numpy as jnp
from jax import lax
from jax.experimental import pallas as pl
from jax.experimental.pallas import tpu as pltpu
from jax.experimental.pallas import tpu_sc as plsc

B, N, L, D = 1, 512, 1024, 64
V_MSA = 22
NC, NS = 2, 16
NW = NC * NS
N_PER_W = N // NW
CHUNK = 256
CPL = L // CHUNK
N_CHUNKS = N_PER_W * CPL
IDX_CHUNK = 128


def _body(msa_hbm, idx_hbm, emb_hbm, pe_hbm, peq_hbm, out_hbm,
          tbl, embv, peqv, idxv, pev, msav, obuf,
          sem_g, sem_a, sem_b):
    wid = lax.axis_index("s") * NC + lax.axis_index("c")
    n0 = wid * N_PER_W

    pltpu.sync_copy(idx_hbm.at[0], idxv)
    gathers = []
    for k in range(L // IDX_CHUNK):
        gathers.append(pltpu.async_copy(
            pe_hbm.at[idxv.at[pl.ds(k * IDX_CHUNK, IDX_CHUNK)]],
            pev.at[pl.ds(k * IDX_CHUNK, IDX_CHUNK)],
            sem_g))

    pltpu.sync_copy(msa_hbm.at[0, pl.ds(n0, N_PER_W)], msav)
    pltpu.sync_copy(emb_hbm, embv)
    pltpu.sync_copy(peq_hbm, peqv)

    peq_regs = [[peqv[s, pl.ds(16 * j, 16)] for j in range(4)] for s in range(2)]
    for s in range(2):
        for i in range(V_MSA):
            for j in range(4):
                tbl[s * V_MSA + i, pl.ds(16 * j, 16)] = (
                    embv[i, pl.ds(16 * j, 16)] + peq_regs[s][j])

    for g in gathers:
        g.wait()

    col16 = lax.iota(jnp.int32, 16)
    sems = [sem_a, sem_b]

    def chunk_do(cc, b):
        n_rel = cc // CPL
        l0 = (cc % CPL) * CHUNK
        ng = n0 + n_rel
        off = jnp.where(ng == 0, 0, V_MSA).astype(jnp.int32)
        ob = obuf.at[b]

        def lblock(lb, _):
            lbase = l0 + lb * 16
            m16 = msav[n_rel, pl.ds(lbase, 16)]
            tvec = m16 + off
            lvec = lbase + col16
            rvec = lb * 16 + col16
            dz = tvec - tvec
            for d in range(D):
                dvec = dz + d
                g = plsc.load_gather(tbl, [tvec, dvec])
                p = plsc.load_gather(pev, [lvec, dvec])
                plsc.store_scatter(ob, [rvec, dvec], g + p)
            return 0

        lax.fori_loop(0, CHUNK // 16, lblock, 0)
        pltpu.async_copy(obuf.at[b], out_hbm.at[0, ng, pl.ds(l0, CHUNK)],
                         sems[b])

    def drain(b):
        pltpu.make_async_copy(obuf.at[b],
                              out_hbm.at[0, 0, pl.ds(0, CHUNK)],
                              sems[b]).wait()

    chunk_do(jnp.int32(0), 0)
    chunk_do(jnp.int32(1), 1)

    def outer(co, _):
        for b in range(2):
            drain(b)
            chunk_do(co * 2 + b, b)
        return 0

    lax.fori_loop(1, N_CHUNKS // 2, outer, 0)
    drain(0)
    drain(1)


@jax.jit
def kernel(msa, idx, emb_W, pe_buf, pe_q):
    mesh = plsc.VectorSubcoreMesh(core_axis_name="c", subcore_axis_name="s",
                                  num_cores=NC, num_subcores=NS)
    fn = pl.kernel(
        _body,
        out_type=jax.ShapeDtypeStruct((B, N, L, D), jnp.float32),
        mesh=mesh,
        scratch_types=[
            pltpu.VMEM((2 * V_MSA, D), jnp.float32),
            pltpu.VMEM((V_MSA, D), jnp.float32),
            pltpu.VMEM((2, D), jnp.float32),
            pltpu.VMEM((L,), jnp.int32),
            pltpu.VMEM((L, D), jnp.float32),
            pltpu.VMEM((N_PER_W, L), jnp.int32),
            pltpu.VMEM((2, CHUNK, D), jnp.float32),
            pltpu.SemaphoreType.DMA,
            pltpu.SemaphoreType.DMA,
            pltpu.SemaphoreType.DMA,
        ],
        compiler_params=pltpu.CompilerParams(needs_layout_passes=False,
                                             use_tc_tiling_on_sc=False),
    )
    return fn(msa, idx, emb_W, pe_buf, pe_q)


# speedup vs baseline: 4.0392x; 4.0392x over previous
import jax
import jax.numpy as jnp
from jax import lax
from jax.experimental import pallas as pl
from jax.experimental.pallas import tpu as pltpu
from jax.experimental.pallas import tpu_sc as plsc

B, N, L, D = 1, 512, 1024, 64
DP = D + 1
V_MSA = 22
NC, NS = 2, 16
NW = NC * NS
N_PER_W = N // NW
CHUNK = 256
CPL = L // CHUNK
N_CHUNKS = N_PER_W * CPL
IDX_CHUNK = 128


def _body(msa_hbm, idx_hbm, emb_hbm, pe_hbm, peq_hbm, out_hbm,
          tbl, embv, peqv, idxv, stage, pev, msav, obuf,
          sem_g, sem_a, sem_b):
    wid = lax.axis_index("s") * NC + lax.axis_index("c")
    n0 = wid * N_PER_W

    pltpu.sync_copy(idx_hbm.at[0], idxv)
    pltpu.sync_copy(msa_hbm.at[0, pl.ds(n0, N_PER_W)], msav)
    pltpu.sync_copy(emb_hbm, embv)
    pltpu.sync_copy(peq_hbm, peqv)

    peq_regs = [[peqv[s, pl.ds(16 * j, 16)] for j in range(4)] for s in range(2)]
    for s in range(2):
        for i in range(V_MSA):
            for j in range(4):
                tbl[pl.ds((s * V_MSA + i) * DP + 16 * j, 16)] = (
                    embv[i, pl.ds(16 * j, 16)] + peq_regs[s][j])

    for k in range(L // IDX_CHUNK):
        pltpu.async_copy(
            pe_hbm.at[idxv.at[pl.ds(k * IDX_CHUNK, IDX_CHUNK)]],
            stage, sem_g).wait()

        def repack(r, _, *, k=k):
            base = (k * IDX_CHUNK + r) * DP
            for j in range(4):
                pev[pl.ds(base + 16 * j, 16)] = stage[r, pl.ds(16 * j, 16)]
            return 0

        lax.fori_loop(0, IDX_CHUNK, repack, 0)

    col16 = lax.iota(jnp.int32, 16)
    sems = [sem_a, sem_b]

    def chunk_do(cc, b):
        n_rel = cc // CPL
        l0 = (cc % CPL) * CHUNK
        ng = n0 + n_rel
        off = jnp.where(ng == 0, 0, V_MSA).astype(jnp.int32)

        def lblock(lb, _):
            lbase = l0 + lb * 16
            m16 = msav[n_rel, pl.ds(lbase, 16)]
            tvec = (m16 + off) * DP
            lvec = (lbase + col16) * DP
            for d in range(D):
                g = plsc.load_gather(tbl, [tvec + d])
                p = plsc.load_gather(pev, [lvec + d])
                obuf[b, d, pl.ds(lb * 16, 16)] = g + p
            return 0

        lax.fori_loop(0, CHUNK // 16, lblock, 0)
        pltpu.async_copy(obuf.at[b], out_hbm.at[ng, :, pl.ds(l0, CHUNK)],
                         sems[b])

    def drain(b):
        pltpu.make_async_copy(obuf.at[b],
                              out_hbm.at[0, :, pl.ds(0, CHUNK)],
                              sems[b]).wait()

    chunk_do(jnp.int32(0), 0)
    chunk_do(jnp.int32(1), 1)

    def outer(co, _):
        for b in range(2):
            drain(b)
            chunk_do(co * 2 + b, b)
        return 0

    lax.fori_loop(1, N_CHUNKS // 2, outer, 0)
    drain(0)
    drain(1)


@jax.jit
def kernel(msa, idx, emb_W, pe_buf, pe_q):
    mesh = plsc.VectorSubcoreMesh(core_axis_name="c", subcore_axis_name="s",
                                  num_cores=NC, num_subcores=NS)
    fn = pl.kernel(
        _body,
        out_type=jax.ShapeDtypeStruct((N, D, L), jnp.float32),
        mesh=mesh,
        scratch_types=[
            pltpu.VMEM((2 * V_MSA * DP,), jnp.float32),
            pltpu.VMEM((V_MSA, D), jnp.float32),
            pltpu.VMEM((2, D), jnp.float32),
            pltpu.VMEM((L,), jnp.int32),
            pltpu.VMEM((IDX_CHUNK, D), jnp.float32),
            pltpu.VMEM((L * DP,), jnp.float32),
            pltpu.VMEM((N_PER_W, L), jnp.int32),
            pltpu.VMEM((2, D, CHUNK), jnp.float32),
            pltpu.SemaphoreType.DMA,
            pltpu.SemaphoreType.DMA,
            pltpu.SemaphoreType.DMA,
        ],
        compiler_params=pltpu.CompilerParams(needs_layout_passes=False,
                                             use_tc_tiling_on_sc=False),
    )
    out_ndl = fn(msa, idx, emb_W, pe_buf, pe_q)
    return jnp.swapaxes(out_ndl, 1, 2)[None]


# speedup vs baseline: 4.1285x; 1.0221x over previous
import jax
import jax.numpy as jnp
from jax import lax
from jax.experimental import pallas as pl
from jax.experimental.pallas import tpu as pltpu
from jax.experimental.pallas import tpu_sc as plsc

B, N, L, D = 1, 512, 1024, 64
DP = D + 1
LP = L + 1
V_MSA = 22
NC, NS = 2, 16
NW = NC * NS
N_PER_W = N // NW
CHUNK = 256
CPL = L // CHUNK
N_CHUNKS = N_PER_W * CPL
IDX_CHUNK = 128


def _body(msa_hbm, idx_hbm, emb_hbm, pe_hbm, peq_hbm, out_hbm,
          tbl, embv, peqv, idxv, stage, pev, msav, obuf,
          sem_g, sem_a, sem_b):
    wid = lax.axis_index("s") * NC + lax.axis_index("c")
    n0 = wid * N_PER_W

    pltpu.sync_copy(idx_hbm.at[0], idxv)
    pltpu.sync_copy(msa_hbm.at[0, pl.ds(n0, N_PER_W)], msav)
    pltpu.sync_copy(emb_hbm, embv)
    pltpu.sync_copy(peq_hbm, peqv)

    peq_regs = [[peqv[s, pl.ds(16 * j, 16)] for j in range(4)] for s in range(2)]
    for s in range(2):
        for i in range(V_MSA):
            for j in range(4):
                tbl[pl.ds((s * V_MSA + i) * DP + 16 * j, 16)] = (
                    embv[i, pl.ds(16 * j, 16)] + peq_regs[s][j])

    col16 = lax.iota(jnp.int32, 16)
    cvecs = [(col16 + 16 * j) * LP for j in range(4)]
    for k in range(L // IDX_CHUNK):
        pltpu.async_copy(
            pe_hbm.at[idxv.at[pl.ds(k * IDX_CHUNK, IDX_CHUNK)]],
            stage, sem_g).wait()

        def repack(r, _, *, k=k):
            lpos = k * IDX_CHUNK + r
            for j in range(4):
                plsc.store_scatter(pev, [cvecs[j] + lpos],
                                   stage[r, pl.ds(16 * j, 16)])
            return 0

        lax.fori_loop(0, IDX_CHUNK, repack, 0)
    sems = [sem_a, sem_b]

    def chunk_do(cc, b):
        n_rel = cc // CPL
        l0 = (cc % CPL) * CHUNK
        ng = n0 + n_rel
        off = jnp.where(ng == 0, 0, V_MSA).astype(jnp.int32)

        def lblock(lb, _):
            lbase = l0 + lb * 16
            m16 = msav[n_rel, pl.ds(lbase, 16)]
            tvec = (m16 + off) * DP
            for d in range(D):
                g = plsc.load_gather(tbl, [tvec + d])
                p = pev[pl.ds(d * LP + lbase, 16)]
                obuf[b, d, pl.ds(lb * 16, 16)] = g + p
            return 0

        lax.fori_loop(0, CHUNK // 16, lblock, 0)
        pltpu.async_copy(obuf.at[b], out_hbm.at[ng, :, pl.ds(l0, CHUNK)],
                         sems[b])

    def drain(b):
        pltpu.make_async_copy(obuf.at[b],
                              out_hbm.at[0, :, pl.ds(0, CHUNK)],
                              sems[b]).wait()

    chunk_do(jnp.int32(0), 0)
    chunk_do(jnp.int32(1), 1)

    def outer(co, _):
        for b in range(2):
            drain(b)
            chunk_do(co * 2 + b, b)
        return 0

    lax.fori_loop(1, N_CHUNKS // 2, outer, 0)
    drain(0)
    drain(1)


@jax.jit
def kernel(msa, idx, emb_W, pe_buf, pe_q):
    mesh = plsc.VectorSubcoreMesh(core_axis_name="c", subcore_axis_name="s",
                                  num_cores=NC, num_subcores=NS)
    fn = pl.kernel(
        _body,
        out_type=jax.ShapeDtypeStruct((N, D, L), jnp.float32),
        mesh=mesh,
        scratch_types=[
            pltpu.VMEM((2 * V_MSA * DP,), jnp.float32),
            pltpu.VMEM((V_MSA, D), jnp.float32),
            pltpu.VMEM((2, D), jnp.float32),
            pltpu.VMEM((L,), jnp.int32),
            pltpu.VMEM((IDX_CHUNK, D), jnp.float32),
            pltpu.VMEM((D * LP,), jnp.float32),
            pltpu.VMEM((N_PER_W, L), jnp.int32),
            pltpu.VMEM((2, D, CHUNK), jnp.float32),
            pltpu.SemaphoreType.DMA,
            pltpu.SemaphoreType.DMA,
            pltpu.SemaphoreType.DMA,
        ],
        compiler_params=pltpu.CompilerParams(needs_layout_passes=False,
                                             use_tc_tiling_on_sc=False),
    )
    out_ndl = fn(msa, idx, emb_W, pe_buf, pe_q)
    return jnp.swapaxes(out_ndl, 1, 2)[None]


# speedup vs baseline: 7.3983x; 1.7920x over previous
import jax
import jax.numpy as jnp
from jax import lax
from jax.experimental import pallas as pl
from jax.experimental.pallas import tpu as pltpu
from jax.experimental.pallas import tpu_sc as plsc

B, N, L, D = 1, 512, 1024, 64
DP = D + 1
LP = L + 1
V_MSA = 22
NC, NS = 2, 16
NW = NC * NS
N_PER_W = N // NW
CHUNK = 256
CPL = L // CHUNK
N_CHUNKS = N_PER_W * CPL
IDX_CHUNK = 128


def _body(msa_hbm, idx_hbm, emb_hbm, pe_hbm, peq_hbm, out_hbm,
          tbl, embv, peqv, idxv, stage, pev, msav, obuf,
          sem_g, sem_a, sem_b):
    wid = lax.axis_index("s") * NC + lax.axis_index("c")
    n0 = wid * N_PER_W

    pltpu.sync_copy(idx_hbm.at[0], idxv)
    pltpu.sync_copy(msa_hbm.at[0, pl.ds(n0, N_PER_W)], msav)
    pltpu.sync_copy(emb_hbm, embv)
    pltpu.sync_copy(peq_hbm, peqv)

    peq_regs = [[peqv[s, pl.ds(16 * j, 16)] for j in range(4)] for s in range(2)]
    for s in range(2):
        for i in range(V_MSA):
            for j in range(4):
                tbl[pl.ds((s * V_MSA + i) * DP + 16 * j, 16)] = (
                    embv[i, pl.ds(16 * j, 16)] + peq_regs[s][j])

    col16 = lax.iota(jnp.int32, 16)
    cvecs = [(col16 + 16 * j) * LP for j in range(4)]
    for k in range(L // IDX_CHUNK):
        pltpu.async_copy(
            pe_hbm.at[idxv.at[pl.ds(k * IDX_CHUNK, IDX_CHUNK)]],
            stage, sem_g).wait()

        def repack(r, _, *, k=k):
            lpos = k * IDX_CHUNK + r
            for j in range(4):
                plsc.store_scatter(pev, [cvecs[j] + lpos],
                                   stage[r, pl.ds(16 * j, 16)])
            return 0

        lax.fori_loop(0, IDX_CHUNK, repack, 0)
    sems = [sem_a, sem_b]

    def chunk_do(cc, b):
        n_rel = cc // CPL
        l0 = (cc % CPL) * CHUNK
        ng = n0 + n_rel
        off = jnp.where(ng == 0, 0, V_MSA).astype(jnp.int32)

        def lblock(lb, _):
            lbase = l0 + lb * 16
            m16 = msav[n_rel, pl.ds(lbase, 16)]
            tvec = (m16 + off) * DP
            for d0 in range(0, D, 8):
                gs = [plsc.load_gather(tbl, [tvec + (d0 + i)])
                      for i in range(8)]
                ps = [pev[pl.ds((d0 + i) * LP + lbase, 16)]
                      for i in range(8)]
                for i in range(8):
                    obuf[b, d0 + i, pl.ds(lb * 16, 16)] = gs[i] + ps[i]
            return 0

        lax.fori_loop(0, CHUNK // 16, lblock, 0)
        pltpu.async_copy(obuf.at[b], out_hbm.at[ng, :, pl.ds(l0, CHUNK)],
                         sems[b])

    def drain(b):
        pltpu.make_async_copy(obuf.at[b],
                              out_hbm.at[0, :, pl.ds(0, CHUNK)],
                              sems[b]).wait()

    chunk_do(jnp.int32(0), 0)
    chunk_do(jnp.int32(1), 1)

    def outer(co, _):
        for b in range(2):
            drain(b)
            chunk_do(co * 2 + b, b)
        return 0

    lax.fori_loop(1, N_CHUNKS // 2, outer, 0)
    drain(0)
    drain(1)


@jax.jit
def kernel(msa, idx, emb_W, pe_buf, pe_q):
    mesh = plsc.VectorSubcoreMesh(core_axis_name="c", subcore_axis_name="s",
                                  num_cores=NC, num_subcores=NS)
    fn = pl.kernel(
        _body,
        out_type=jax.ShapeDtypeStruct((N, D, L), jnp.float32),
        mesh=mesh,
        scratch_types=[
            pltpu.VMEM((2 * V_MSA * DP,), jnp.float32),
            pltpu.VMEM((V_MSA, D), jnp.float32),
            pltpu.VMEM((2, D), jnp.float32),
            pltpu.VMEM((L,), jnp.int32),
            pltpu.VMEM((IDX_CHUNK, D), jnp.float32),
            pltpu.VMEM((D * LP,), jnp.float32),
            pltpu.VMEM((N_PER_W, L), jnp.int32),
            pltpu.VMEM((2, D, CHUNK), jnp.float32),
            pltpu.SemaphoreType.DMA,
            pltpu.SemaphoreType.DMA,
            pltpu.SemaphoreType.DMA,
        ],
        compiler_params=pltpu.CompilerParams(needs_layout_passes=False,
                                             use_tc_tiling_on_sc=False),
    )
    out_ndl = fn(msa, idx, emb_W, pe_buf, pe_q)
    return jnp.swapaxes(out_ndl, 1, 2)[None]
